# baseline (speedup 1.0000x reference)
import functools

import jax
import jax.numpy as jnp
from jax import lax
from jax.experimental import pallas as pl
from jax.experimental.pallas import tpu as pltpu
from jax.experimental.pallas import tpu_sc as plsc

N_INST = 10000
N_NET = 2000
N = N_INST + N_NET
E = 320000
D = 128
NC = 2
NS = 16
N_PAD = 12288
ROWS_PER_TILE = N_PAD // NS
CHUNK = 80
E_PER_W = E // NS
N_CHUNKS = E_PER_W // CHUNK
E_DIR = E
ZROWS = 8


def _leaky(v):
    return jnp.where(v >= 0, v, 0.1 * v)


def _deg_body(dst_hbm, out_hbm, hist_v, idx_v):
    c = lax.axis_index("c")
    s = lax.axis_index("s")
    zeros16 = jnp.zeros((16,), jnp.float32)
    ones16 = jnp.ones((16,), jnp.float32)

    def zero_step(i, _):
        hist_v[pl.ds(i * 16, 16)] = zeros16
        return 0

    lax.fori_loop(0, N_PAD // 16, zero_step, 0)

    pltpu.sync_copy(dst_hbm.at[pl.ds(c * E_DIR + s * E_PER_W, E_PER_W)], idx_v)

    def inner(j, _):
        idx = idx_v[pl.ds(j * 16, 16)]
        plsc.addupdate_scatter(hist_v, [idx], ones16)
        return 0

    lax.fori_loop(0, E_PER_W // 16, inner, 0)
    pltpu.sync_copy(hist_v, out_hbm.at[pl.ds((c * NS + s) * N_PAD, N_PAD)])


@jax.jit
def _deg_kernel(dst_flat):
    mesh = plsc.VectorSubcoreMesh(core_axis_name="c", subcore_axis_name="s")
    return pl.kernel(
        _deg_body,
        out_type=jax.ShapeDtypeStruct((NC * NS * N_PAD,), jnp.float32),
        mesh=mesh,
        scratch_types=[
            pltpu.VMEM((N_PAD,), jnp.float32),
            pltpu.VMEM((E_PER_W,), jnp.int32),
        ],
        compiler_params=pltpu.CompilerParams(needs_layout_passes=False),
    )(dst_flat)


def _edge_body(g2_hbm, src_hbm, dst_hbm, out_hbm, srcA_v, srcB_v, srcC_v,
               dstA_v, dstB_v, dstC_v, rows0_v, rows1_v, rows2_v, zero_v,
               acc_sh, gsem0, gsem1, gsem2, ssem0, ssem1, ssem2, isem0,
               isem1, isem2, zsem):
    c = lax.axis_index("c")
    s = lax.axis_index("s")
    zeros16 = jnp.zeros((16,), jnp.float32)
    lane = lax.iota(jnp.int32, 16)

    def zero_step(i, _):
        r = jnp.full((16,), i // (D // 16), jnp.int32)
        col = (i % (D // 16)) * 16 + lane
        plsc.store_scatter(zero_v, [r, col], zeros16)
        return 0

    lax.fori_loop(0, ZROWS * (D // 16), zero_step, 0)

    def zfill(i, _):
        pltpu.async_copy(
            zero_v, acc_sh.at[pl.ds(s * ROWS_PER_TILE + i * ZROWS, ZROWS)], zsem)
        return 0

    lax.fori_loop(0, ROWS_PER_TILE // ZROWS, zfill, 0)

    def zdrain(i, _):
        pltpu.make_async_copy(
            zero_v, acc_sh.at[pl.ds(s * ROWS_PER_TILE + i * ZROWS, ZROWS)], zsem
        ).wait()
        return 0

    lax.fori_loop(0, ROWS_PER_TILE // ZROWS, zdrain, 0)
    plsc.subcore_barrier()

    base0 = c * E_DIR + s * E_PER_W
    rows = (rows0_v, rows1_v, rows2_v)
    srcs = (srcA_v, srcB_v, srcC_v)
    dsts = (dstA_v, dstB_v, dstC_v)
    gsems = (gsem0, gsem1, gsem2)
    ssems = (ssem0, ssem1, ssem2)
    isems = (isem0, isem1, isem2)

    def stage_idx(chunk, b):
        pltpu.async_copy(src_hbm.at[pl.ds(base0 + chunk * CHUNK, CHUNK)],
                         srcs[b], isems[b])
        pltpu.async_copy(dst_hbm.at[pl.ds(base0 + chunk * CHUNK, CHUNK)],
                         dsts[b], isems[b])

    def wait_idx(b):
        pltpu.make_async_copy(src_hbm.at[pl.ds(0, CHUNK)], srcs[b], isems[b]).wait()
        pltpu.make_async_copy(dst_hbm.at[pl.ds(0, CHUNK)], dsts[b], isems[b]).wait()

    def gather(b):
        pltpu.async_copy(g2_hbm.at[srcs[b]], rows[b], gsems[b])

    def wait_gather(b):
        pltpu.make_async_copy(g2_hbm.at[srcs[b]], rows[b], gsems[b]).wait()

    def scatter(b):
        pltpu.async_copy(rows[b], acc_sh.at[dsts[b]], ssems[b], add=True)

    def wait_scatter(b):
        pltpu.make_async_copy(rows[b], acc_sh.at[dsts[b]], ssems[b]).wait()

    pltpu.sync_copy(src_hbm.at[pl.ds(base0, CHUNK)], srcA_v)
    pltpu.sync_copy(dst_hbm.at[pl.ds(base0, CHUNK)], dstA_v)
    gather(0)
    stage_idx(1, 1)
    wait_gather(0)
    scatter(0)
    wait_idx(1)
    gather(1)
    stage_idx(2, 2)

    def outer(p, _):
        for o in (1, 2, 3):
            j = o % 3
            j1 = (o + 1) % 3
            j2 = (o + 2) % 3
            i = 3 * p + o
            wait_gather(j)
            scatter(j)
            wait_scatter(j2)
            wait_idx(j1)
            gather(j1)
            stage_idx(jnp.minimum(i + 2, N_CHUNKS - 1), j2)
        return 0

    lax.fori_loop(0, (N_CHUNKS - 1) // 3, outer, 0)
    wait_scatter(0)
    wait_gather(1)
    wait_idx(2)
    plsc.subcore_barrier()

    pltpu.sync_copy(
        acc_sh.at[pl.ds(s * ROWS_PER_TILE, ROWS_PER_TILE)],
        out_hbm.at[pl.ds(c * N_PAD + s * ROWS_PER_TILE, ROWS_PER_TILE)],
    )


@jax.jit
def _edge_kernel(g2, src_flat, dst_flat):
    mesh = plsc.VectorSubcoreMesh(core_axis_name="c", subcore_axis_name="s")
    return pl.kernel(
        _edge_body,
        out_type=jax.ShapeDtypeStruct((NC * N_PAD, D), jnp.float32),
        mesh=mesh,
        scratch_types=[
            pltpu.VMEM((CHUNK,), jnp.int32),
            pltpu.VMEM((CHUNK,), jnp.int32),
            pltpu.VMEM((CHUNK,), jnp.int32),
            pltpu.VMEM((CHUNK,), jnp.int32),
            pltpu.VMEM((CHUNK,), jnp.int32),
            pltpu.VMEM((CHUNK,), jnp.int32),
            pltpu.VMEM((CHUNK, D), jnp.float32),
            pltpu.VMEM((CHUNK, D), jnp.float32),
            pltpu.VMEM((CHUNK, D), jnp.float32),
            pltpu.VMEM((ZROWS, D), jnp.float32),
            pltpu.VMEM_SHARED((N_PAD, D), jnp.float32),
            pltpu.SemaphoreType.DMA,
            pltpu.SemaphoreType.DMA,
            pltpu.SemaphoreType.DMA,
            pltpu.SemaphoreType.DMA,
            pltpu.SemaphoreType.DMA,
            pltpu.SemaphoreType.DMA,
            pltpu.SemaphoreType.DMA,
            pltpu.SemaphoreType.DMA,
            pltpu.SemaphoreType.DMA,
            pltpu.SemaphoreType.DMA,
        ],
        compiler_params=pltpu.CompilerParams(needs_layout_passes=False),
    )(g2, src_flat, dst_flat)


def _mlp_body(x_ref, w1_ref, b1_ref, w2_ref, b2_ref, o_ref):
    h1 = _leaky(
        jnp.dot(x_ref[...], w1_ref[...], preferred_element_type=jnp.float32)
        + b1_ref[...]
    )
    o_ref[...] = _leaky(
        jnp.dot(h1, w2_ref[...], preferred_element_type=jnp.float32) + b2_ref[...]
    )


def _mlp(x, w1, b1, w2, b2, blk):
    m, _ = x.shape
    d_out = w2.shape[1]
    grid = m // blk
    return pl.pallas_call(
        _mlp_body,
        grid=(grid,),
        in_specs=[
            pl.BlockSpec((blk, x.shape[1]), lambda i: (i, 0)),
            pl.BlockSpec(w1.shape, lambda i: (0, 0)),
            pl.BlockSpec((1, b1.shape[0]), lambda i: (0, 0)),
            pl.BlockSpec(w2.shape, lambda i: (0, 0)),
            pl.BlockSpec((1, b2.shape[0]), lambda i: (0, 0)),
        ],
        out_specs=pl.BlockSpec((blk, d_out), lambda i: (i, 0)),
        out_shape=jax.ShapeDtypeStruct((m, d_out), jnp.float32),
    )(x, w1, b1[None, :], w2, b2[None, :])


def _stats_body(hist_ref, o_ref):
    deg_col = jnp.sum(hist_ref[0], axis=0, keepdims=True) + 1.0
    deg_row = jnp.sum(hist_ref[1], axis=0, keepdims=True) + 1.0
    o_ref[0:1] = lax.rsqrt(deg_row)
    o_ref[1:2] = 1.0 / deg_row
    o_ref[2:3] = lax.rsqrt(deg_col)
    o_ref[3:4] = 1.0 / deg_col


@jax.jit
def _stats_kernel(hist):
    return pl.pallas_call(
        _stats_body,
        out_shape=jax.ShapeDtypeStruct((4, N_PAD), jnp.float32),
    )(hist)


def _prep_body(h_ref, wf_ref, bf_ref, rootf_ref, wr_ref, br_ref, rootr_ref,
               sc_ref, g2_ref, self_ref):
    h = h_ref[...]
    dis_row = sc_ref[0, :, :]
    inv_row = sc_ref[1, :, :]
    dis_col = sc_ref[2, :, :]
    inv_col = sc_ref[3, :, :]
    xf = jnp.dot(h, wf_ref[...], preferred_element_type=jnp.float32) + bf_ref[...]
    xr = jnp.dot(h, wr_ref[...], preferred_element_type=jnp.float32) + br_ref[...]
    g2_ref[0] = dis_row * jax.nn.relu(xf)
    g2_ref[1] = dis_col * jax.nn.relu(xr)
    self_ref[...] = (
        inv_row * jax.nn.relu(xf + rootf_ref[...])
        + inv_col * jax.nn.relu(xr + rootr_ref[...])
    )


def _prep(h, wf, bf, rootf, wr, br, rootr, sc4, blk):
    grid = N // blk
    g2, selfsum = pl.pallas_call(
        _prep_body,
        grid=(grid,),
        in_specs=[
            pl.BlockSpec((blk, D), lambda i: (i, 0)),
            pl.BlockSpec((D, D), lambda i: (0, 0)),
            pl.BlockSpec((1, D), lambda i: (0, 0)),
            pl.BlockSpec((1, D), lambda i: (0, 0)),
            pl.BlockSpec((D, D), lambda i: (0, 0)),
            pl.BlockSpec((1, D), lambda i: (0, 0)),
            pl.BlockSpec((1, D), lambda i: (0, 0)),
            pl.BlockSpec((4, blk, 1), lambda i: (0, i, 0)),
        ],
        out_specs=[
            pl.BlockSpec((2, blk, D), lambda i: (0, i, 0)),
            pl.BlockSpec((blk, D), lambda i: (i, 0)),
        ],
        out_shape=[
            jax.ShapeDtypeStruct((2, N, D), jnp.float32),
            jax.ShapeDtypeStruct((N, D), jnp.float32),
        ],
    )(h, wf, bf[None, :], rootf[None, :], wr, br[None, :], rootr[None, :], sc4)
    return g2, selfsum


def _post_body(aggf_ref, aggr_ref, sc_ref, self_ref, h_ref, g_ref, b_ref,
               o_ref, *, leaky):
    dis_row = sc_ref[0, :, :]
    dis_col = sc_ref[2, :, :]
    hh = dis_row * aggf_ref[0] + dis_col * aggr_ref[0] + self_ref[...]
    mu = jnp.mean(hh, axis=-1, keepdims=True)
    var = jnp.mean(hh * hh, axis=-1, keepdims=True) - mu * mu
    hh = (hh - mu) * lax.rsqrt(var + 1e-5) * g_ref[...] + b_ref[...]
    if leaky:
        hh = _leaky(hh)
    o_ref[...] = hh + h_ref[...]


def _post(agg, sc4, selfsum, h, ln_g, ln_b, leaky, blk):
    grid = N // blk
    return pl.pallas_call(
        functools.partial(_post_body, leaky=leaky),
        grid=(grid,),
        in_specs=[
            pl.BlockSpec((1, blk, D), lambda i: (0, i, 0)),
            pl.BlockSpec((1, blk, D), lambda i: (1, i, 0)),
            pl.BlockSpec((4, blk, 1), lambda i: (0, i, 0)),
            pl.BlockSpec((blk, D), lambda i: (i, 0)),
            pl.BlockSpec((blk, D), lambda i: (i, 0)),
            pl.BlockSpec((1, D), lambda i: (0, 0)),
            pl.BlockSpec((1, D), lambda i: (0, 0)),
        ],
        out_specs=pl.BlockSpec((blk, D), lambda i: (i, 0)),
        out_shape=jax.ShapeDtypeStruct((N, D), jnp.float32),
    )(agg, agg, sc4, selfsum, h, ln_g[None, :], ln_b[None, :])


def _postprep_body(aggf_ref, aggr_ref, sc_ref, self_ref, h_ref, g_ref, b_ref,
                   wf_ref, bf_ref, rootf_ref, wr_ref, br_ref, rootr_ref,
                   o_ref, g2_ref, self2_ref, *, leaky):
    dis_row = sc_ref[0, :, :]
    inv_row = sc_ref[1, :, :]
    dis_col = sc_ref[2, :, :]
    inv_col = sc_ref[3, :, :]
    hh = dis_row * aggf_ref[0] + dis_col * aggr_ref[0] + self_ref[...]
    mu = jnp.mean(hh, axis=-1, keepdims=True)
    var = jnp.mean(hh * hh, axis=-1, keepdims=True) - mu * mu
    hh = (hh - mu) * lax.rsqrt(var + 1e-5) * g_ref[...] + b_ref[...]
    if leaky:
        hh = _leaky(hh)
    h = hh + h_ref[...]
    o_ref[...] = h
    xf = jnp.dot(h, wf_ref[...], preferred_element_type=jnp.float32) + bf_ref[...]
    xr = jnp.dot(h, wr_ref[...], preferred_element_type=jnp.float32) + br_ref[...]
    g2_ref[0] = dis_row * jax.nn.relu(xf)
    g2_ref[1] = dis_col * jax.nn.relu(xr)
    self2_ref[...] = (
        inv_row * jax.nn.relu(xf + rootf_ref[...])
        + inv_col * jax.nn.relu(xr + rootr_ref[...])
    )


def _postprep(agg, sc4, selfsum, h, ln_g, ln_b, wf, bf, rootf, wr, br, rootr,
              leaky, blk):
    grid = N // blk
    full = lambda i: (0, 0)
    return pl.pallas_call(
        functools.partial(_postprep_body, leaky=leaky),
        grid=(grid,),
        in_specs=[
            pl.BlockSpec((1, blk, D), lambda i: (0, i, 0)),
            pl.BlockSpec((1, blk, D), lambda i: (1, i, 0)),
            pl.BlockSpec((4, blk, 1), lambda i: (0, i, 0)),
            pl.BlockSpec((blk, D), lambda i: (i, 0)),
            pl.BlockSpec((blk, D), lambda i: (i, 0)),
            pl.BlockSpec((1, D), full),
            pl.BlockSpec((1, D), full),
            pl.BlockSpec((D, D), full),
            pl.BlockSpec((1, D), full),
            pl.BlockSpec((1, D), full),
            pl.BlockSpec((D, D), full),
            pl.BlockSpec((1, D), full),
            pl.BlockSpec((1, D), full),
        ],
        out_specs=[
            pl.BlockSpec((blk, D), lambda i: (i, 0)),
            pl.BlockSpec((2, blk, D), lambda i: (0, i, 0)),
            pl.BlockSpec((blk, D), lambda i: (i, 0)),
        ],
        out_shape=[
            jax.ShapeDtypeStruct((N, D), jnp.float32),
            jax.ShapeDtypeStruct((2, N, D), jnp.float32),
            jax.ShapeDtypeStruct((N, D), jnp.float32),
        ],
    )(agg, agg, sc4, selfsum, h, ln_g[None, :], ln_b[None, :], wf, bf[None, :],
      rootf[None, :], wr, br[None, :], rootr[None, :])


@jax.jit
def kernel(x, x_net, edge_index, enc_W1, enc_b1, enc_W2, enc_b2, net_W1,
           net_b1, net_W2, net_b2, conv_W, conv_b, conv_root, reconv_W,
           reconv_b, reconv_root, ln_g, ln_b):
    row = edge_index[0]
    col = edge_index[1]
    src_flat = jnp.concatenate([row, col + N])
    dst_flat = jnp.concatenate([col, row])

    hist = _deg_kernel(dst_flat).reshape(NC, NS, N_PAD)
    sc4 = _stats_kernel(hist)
    sc4 = sc4[:, :N].reshape(4, N, 1)

    h_inst = _mlp(x, enc_W1, enc_b1, enc_W2, enc_b2, blk=1000)
    h_net = _mlp(x_net, net_W1, net_b1, net_W2, net_b2, blk=1000)
    h = jnp.concatenate([h_inst, h_net], axis=0)

    h0 = h
    g2, selfsum = _prep(h0, conv_W[0], conv_b[0], conv_root[0],
                        reconv_W[0], reconv_b[0], reconv_root[0],
                        sc4, blk=1200)
    agg = _edge_kernel(g2.reshape(2 * N, D), src_flat, dst_flat)
    agg = agg.reshape(2, N_PAD, D)
    h1, g2, selfsum = _postprep(agg, sc4, selfsum, h0, ln_g[0], ln_b[0],
                                conv_W[1], conv_b[1], conv_root[1],
                                reconv_W[1], reconv_b[1], reconv_root[1],
                                leaky=True, blk=1200)
    agg = _edge_kernel(g2.reshape(2 * N, D), src_flat, dst_flat)
    agg = agg.reshape(2, N_PAD, D)
    h2 = _post(agg, sc4, selfsum, h1, ln_g[1], ln_b[1], leaky=False, blk=1200)
    return jnp.concatenate([h0, h1, h2], axis=1)


# speedup vs baseline: 1.0584x; 1.0584x over previous
import functools

import jax
import jax.numpy as jnp
from jax import lax
from jax.experimental import pallas as pl
from jax.experimental.pallas import tpu as pltpu
from jax.experimental.pallas import tpu_sc as plsc

N_INST = 10000
N_NET = 2000
N = N_INST + N_NET
E = 320000
D = 128
NC = 2
NS = 16
N_PAD = 12288
ROWS_PER_TILE = N_PAD // NS
CHUNK = 80
E_PER_W = E // NS
N_CHUNKS = E_PER_W // CHUNK
E_DIR = E
ZROWS = 16


def _leaky(v):
    return jnp.where(v >= 0, v, 0.1 * v)


def _deg_body(dst_hbm, out_hbm, hist_v, idx_v):
    c = lax.axis_index("c")
    s = lax.axis_index("s")
    zeros16 = jnp.zeros((16,), jnp.float32)
    ones16 = jnp.ones((16,), jnp.float32)

    def zero_step(i, _):
        hist_v[pl.ds(i * 16, 16)] = zeros16
        return 0

    lax.fori_loop(0, N_PAD // 16, zero_step, 0)

    pltpu.sync_copy(dst_hbm.at[pl.ds(c * E_DIR + s * E_PER_W, E_PER_W)], idx_v)

    def inner(j, _):
        idx = idx_v[pl.ds(j * 16, 16)]
        plsc.addupdate_scatter(hist_v, [idx], ones16)
        return 0

    lax.fori_loop(0, E_PER_W // 16, inner, 0)
    pltpu.sync_copy(hist_v, out_hbm.at[pl.ds((c * NS + s) * N_PAD, N_PAD)])


@jax.jit
def _deg_kernel(dst_flat):
    mesh = plsc.VectorSubcoreMesh(core_axis_name="c", subcore_axis_name="s")
    return pl.kernel(
        _deg_body,
        out_type=jax.ShapeDtypeStruct((NC * NS * N_PAD,), jnp.float32),
        mesh=mesh,
        scratch_types=[
            pltpu.VMEM((N_PAD,), jnp.float32),
            pltpu.VMEM((E_PER_W,), jnp.int32),
        ],
        compiler_params=pltpu.CompilerParams(needs_layout_passes=False),
    )(dst_flat)


def _edge_body(g2_hbm, src_hbm, dst_hbm, out_hbm, srcA_v, srcB_v, dstA_v,
               dstB_v, rows0_v, rows1_v, zero_v, acc_sh, gsem0, gsem1,
               isem0, isem1, zsem):
    c = lax.axis_index("c")
    s = lax.axis_index("s")
    zeros16 = jnp.zeros((16,), jnp.float32)
    lane = lax.iota(jnp.int32, 16)

    def zero_step(i, _):
        r = jnp.full((16,), i // (D // 16), jnp.int32)
        col = (i % (D // 16)) * 16 + lane
        plsc.store_scatter(zero_v, [r, col], zeros16)
        return 0

    lax.fori_loop(0, ZROWS * (D // 16), zero_step, 0)

    def zfill(i, _):
        pltpu.async_copy(
            zero_v, acc_sh.at[pl.ds(s * ROWS_PER_TILE + i * ZROWS, ZROWS)], zsem)
        return 0

    lax.fori_loop(0, ROWS_PER_TILE // ZROWS, zfill, 0)

    def zdrain(i, _):
        pltpu.make_async_copy(
            zero_v, acc_sh.at[pl.ds(s * ROWS_PER_TILE + i * ZROWS, ZROWS)], zsem
        ).wait()
        return 0

    lax.fori_loop(0, ROWS_PER_TILE // ZROWS, zdrain, 0)
    plsc.subcore_barrier()

    base0 = c * E_DIR + s * E_PER_W
    rows = (rows0_v, rows1_v)
    srcs = (srcA_v, srcB_v)
    dsts = (dstA_v, dstB_v)
    gsems = (gsem0, gsem1)
    isems = (isem0, isem1)

    def stage_idx(chunk, b):
        pltpu.async_copy(src_hbm.at[pl.ds(base0 + chunk * CHUNK, CHUNK)],
                         srcs[b], isems[b])
        pltpu.async_copy(dst_hbm.at[pl.ds(base0 + chunk * CHUNK, CHUNK)],
                         dsts[b], isems[b])

    def wait_idx(b):
        pltpu.make_async_copy(src_hbm.at[pl.ds(0, CHUNK)], srcs[b], isems[b]).wait()
        pltpu.make_async_copy(dst_hbm.at[pl.ds(0, CHUNK)], dsts[b], isems[b]).wait()

    pltpu.sync_copy(src_hbm.at[pl.ds(base0, CHUNK)], srcA_v)
    pltpu.sync_copy(dst_hbm.at[pl.ds(base0, CHUNK)], dstA_v)
    pltpu.async_copy(g2_hbm.at[srcA_v], rows0_v, gsem0)
    stage_idx(1, 1)

    def outer(p, _):
        for b in range(2):
            i = p * 2 + b
            nb = 1 - b
            wait_idx(nb)
            pltpu.async_copy(g2_hbm.at[srcs[nb]], rows[nb], gsems[nb])
            pltpu.make_async_copy(g2_hbm.at[srcs[b]], rows[b], gsems[b]).wait()
            pltpu.sync_copy(rows[b], acc_sh.at[dsts[b]], add=True)
            nxt = jnp.minimum(i + 2, N_CHUNKS - 1)
            stage_idx(nxt, b)
        return 0

    lax.fori_loop(0, N_CHUNKS // 2, outer, 0)
    pltpu.make_async_copy(g2_hbm.at[srcA_v], rows0_v, gsem0).wait()
    wait_idx(1)
    plsc.subcore_barrier()

    pltpu.sync_copy(
        acc_sh.at[pl.ds(s * ROWS_PER_TILE, ROWS_PER_TILE)],
        out_hbm.at[pl.ds(c * N_PAD + s * ROWS_PER_TILE, ROWS_PER_TILE)],
    )


@jax.jit
def _edge_kernel(g2, src_flat, dst_flat):
    mesh = plsc.VectorSubcoreMesh(core_axis_name="c", subcore_axis_name="s")
    return pl.kernel(
        _edge_body,
        out_type=jax.ShapeDtypeStruct((NC * N_PAD, D), jnp.float32),
        mesh=mesh,
        scratch_types=[
            pltpu.VMEM((CHUNK,), jnp.int32),
            pltpu.VMEM((CHUNK,), jnp.int32),
            pltpu.VMEM((CHUNK,), jnp.int32),
            pltpu.VMEM((CHUNK,), jnp.int32),
            pltpu.VMEM((CHUNK, D), jnp.float32),
            pltpu.VMEM((CHUNK, D), jnp.float32),
            pltpu.VMEM((ZROWS, D), jnp.float32),
            pltpu.VMEM_SHARED((N_PAD, D), jnp.float32),
            pltpu.SemaphoreType.DMA,
            pltpu.SemaphoreType.DMA,
            pltpu.SemaphoreType.DMA,
            pltpu.SemaphoreType.DMA,
            pltpu.SemaphoreType.DMA,
        ],
        compiler_params=pltpu.CompilerParams(needs_layout_passes=False),
    )(g2, src_flat, dst_flat)


def _mlp_body(x_ref, w1_ref, b1_ref, w2_ref, b2_ref, o_ref):
    h1 = _leaky(
        jnp.dot(x_ref[...], w1_ref[...], preferred_element_type=jnp.float32)
        + b1_ref[...]
    )
    o_ref[...] = _leaky(
        jnp.dot(h1, w2_ref[...], preferred_element_type=jnp.float32) + b2_ref[...]
    )


def _mlp(x, w1, b1, w2, b2, blk):
    m, _ = x.shape
    d_out = w2.shape[1]
    grid = m // blk
    return pl.pallas_call(
        _mlp_body,
        grid=(grid,),
        in_specs=[
            pl.BlockSpec((blk, x.shape[1]), lambda i: (i, 0)),
            pl.BlockSpec(w1.shape, lambda i: (0, 0)),
            pl.BlockSpec((1, b1.shape[0]), lambda i: (0, 0)),
            pl.BlockSpec(w2.shape, lambda i: (0, 0)),
            pl.BlockSpec((1, b2.shape[0]), lambda i: (0, 0)),
        ],
        out_specs=pl.BlockSpec((blk, d_out), lambda i: (i, 0)),
        out_shape=jax.ShapeDtypeStruct((m, d_out), jnp.float32),
    )(x, w1, b1[None, :], w2, b2[None, :])


def _stats_body(hist_ref, o_ref):
    deg_col = jnp.sum(hist_ref[0], axis=0, keepdims=True) + 1.0
    deg_row = jnp.sum(hist_ref[1], axis=0, keepdims=True) + 1.0
    o_ref[0:1] = lax.rsqrt(deg_row)
    o_ref[1:2] = 1.0 / deg_row
    o_ref[2:3] = lax.rsqrt(deg_col)
    o_ref[3:4] = 1.0 / deg_col


@jax.jit
def _stats_kernel(hist):
    return pl.pallas_call(
        _stats_body,
        out_shape=jax.ShapeDtypeStruct((4, N_PAD), jnp.float32),
    )(hist)


def _prep_body(h_ref, wf_ref, bf_ref, rootf_ref, wr_ref, br_ref, rootr_ref,
               sc_ref, g2_ref, self_ref):
    h = h_ref[...]
    dis_row = sc_ref[0, :, :]
    inv_row = sc_ref[1, :, :]
    dis_col = sc_ref[2, :, :]
    inv_col = sc_ref[3, :, :]
    xf = jnp.dot(h, wf_ref[...], preferred_element_type=jnp.float32) + bf_ref[...]
    xr = jnp.dot(h, wr_ref[...], preferred_element_type=jnp.float32) + br_ref[...]
    g2_ref[0] = dis_row * jax.nn.relu(xf)
    g2_ref[1] = dis_col * jax.nn.relu(xr)
    self_ref[...] = (
        inv_row * jax.nn.relu(xf + rootf_ref[...])
        + inv_col * jax.nn.relu(xr + rootr_ref[...])
    )


def _prep(h, wf, bf, rootf, wr, br, rootr, sc4, blk):
    grid = N // blk
    g2, selfsum = pl.pallas_call(
        _prep_body,
        grid=(grid,),
        in_specs=[
            pl.BlockSpec((blk, D), lambda i: (i, 0)),
            pl.BlockSpec((D, D), lambda i: (0, 0)),
            pl.BlockSpec((1, D), lambda i: (0, 0)),
            pl.BlockSpec((1, D), lambda i: (0, 0)),
            pl.BlockSpec((D, D), lambda i: (0, 0)),
            pl.BlockSpec((1, D), lambda i: (0, 0)),
            pl.BlockSpec((1, D), lambda i: (0, 0)),
            pl.BlockSpec((4, blk, 1), lambda i: (0, i, 0)),
        ],
        out_specs=[
            pl.BlockSpec((2, blk, D), lambda i: (0, i, 0)),
            pl.BlockSpec((blk, D), lambda i: (i, 0)),
        ],
        out_shape=[
            jax.ShapeDtypeStruct((2, N, D), jnp.float32),
            jax.ShapeDtypeStruct((N, D), jnp.float32),
        ],
    )(h, wf, bf[None, :], rootf[None, :], wr, br[None, :], rootr[None, :], sc4)
    return g2, selfsum


def _post_body(aggf_ref, aggr_ref, sc_ref, self_ref, h_ref, g_ref, b_ref,
               o_ref, *, leaky):
    dis_row = sc_ref[0, :, :]
    dis_col = sc_ref[2, :, :]
    hh = dis_row * aggf_ref[0] + dis_col * aggr_ref[0] + self_ref[...]
    mu = jnp.mean(hh, axis=-1, keepdims=True)
    var = jnp.mean(hh * hh, axis=-1, keepdims=True) - mu * mu
    hh = (hh - mu) * lax.rsqrt(var + 1e-5) * g_ref[...] + b_ref[...]
    if leaky:
        hh = _leaky(hh)
    o_ref[...] = hh + h_ref[...]


def _post(agg, sc4, selfsum, h, ln_g, ln_b, leaky, blk):
    grid = N // blk
    return pl.pallas_call(
        functools.partial(_post_body, leaky=leaky),
        grid=(grid,),
        in_specs=[
            pl.BlockSpec((1, blk, D), lambda i: (0, i, 0)),
            pl.BlockSpec((1, blk, D), lambda i: (1, i, 0)),
            pl.BlockSpec((4, blk, 1), lambda i: (0, i, 0)),
            pl.BlockSpec((blk, D), lambda i: (i, 0)),
            pl.BlockSpec((blk, D), lambda i: (i, 0)),
            pl.BlockSpec((1, D), lambda i: (0, 0)),
            pl.BlockSpec((1, D), lambda i: (0, 0)),
        ],
        out_specs=pl.BlockSpec((blk, D), lambda i: (i, 0)),
        out_shape=jax.ShapeDtypeStruct((N, D), jnp.float32),
    )(agg, agg, sc4, selfsum, h, ln_g[None, :], ln_b[None, :])


def _postprep_body(aggf_ref, aggr_ref, sc_ref, self_ref, h_ref, g_ref, b_ref,
                   wf_ref, bf_ref, rootf_ref, wr_ref, br_ref, rootr_ref,
                   o_ref, g2_ref, self2_ref, *, leaky):
    dis_row = sc_ref[0, :, :]
    inv_row = sc_ref[1, :, :]
    dis_col = sc_ref[2, :, :]
    inv_col = sc_ref[3, :, :]
    hh = dis_row * aggf_ref[0] + dis_col * aggr_ref[0] + self_ref[...]
    mu = jnp.mean(hh, axis=-1, keepdims=True)
    var = jnp.mean(hh * hh, axis=-1, keepdims=True) - mu * mu
    hh = (hh - mu) * lax.rsqrt(var + 1e-5) * g_ref[...] + b_ref[...]
    if leaky:
        hh = _leaky(hh)
    h = hh + h_ref[...]
    o_ref[...] = h
    xf = jnp.dot(h, wf_ref[...], preferred_element_type=jnp.float32) + bf_ref[...]
    xr = jnp.dot(h, wr_ref[...], preferred_element_type=jnp.float32) + br_ref[...]
    g2_ref[0] = dis_row * jax.nn.relu(xf)
    g2_ref[1] = dis_col * jax.nn.relu(xr)
    self2_ref[...] = (
        inv_row * jax.nn.relu(xf + rootf_ref[...])
        + inv_col * jax.nn.relu(xr + rootr_ref[...])
    )


def _postprep(agg, sc4, selfsum, h, ln_g, ln_b, wf, bf, rootf, wr, br, rootr,
              leaky, blk):
    grid = N // blk
    full = lambda i: (0, 0)
    return pl.pallas_call(
        functools.partial(_postprep_body, leaky=leaky),
        grid=(grid,),
        in_specs=[
            pl.BlockSpec((1, blk, D), lambda i: (0, i, 0)),
            pl.BlockSpec((1, blk, D), lambda i: (1, i, 0)),
            pl.BlockSpec((4, blk, 1), lambda i: (0, i, 0)),
            pl.BlockSpec((blk, D), lambda i: (i, 0)),
            pl.BlockSpec((blk, D), lambda i: (i, 0)),
            pl.BlockSpec((1, D), full),
            pl.BlockSpec((1, D), full),
            pl.BlockSpec((D, D), full),
            pl.BlockSpec((1, D), full),
            pl.BlockSpec((1, D), full),
            pl.BlockSpec((D, D), full),
            pl.BlockSpec((1, D), full),
            pl.BlockSpec((1, D), full),
        ],
        out_specs=[
            pl.BlockSpec((blk, D), lambda i: (i, 0)),
            pl.BlockSpec((2, blk, D), lambda i: (0, i, 0)),
            pl.BlockSpec((blk, D), lambda i: (i, 0)),
        ],
        out_shape=[
            jax.ShapeDtypeStruct((N, D), jnp.float32),
            jax.ShapeDtypeStruct((2, N, D), jnp.float32),
            jax.ShapeDtypeStruct((N, D), jnp.float32),
        ],
    )(agg, agg, sc4, selfsum, h, ln_g[None, :], ln_b[None, :], wf, bf[None, :],
      rootf[None, :], wr, br[None, :], rootr[None, :])


@jax.jit
def kernel(x, x_net, edge_index, enc_W1, enc_b1, enc_W2, enc_b2, net_W1,
           net_b1, net_W2, net_b2, conv_W, conv_b, conv_root, reconv_W,
           reconv_b, reconv_root, ln_g, ln_b):
    row = edge_index[0]
    col = edge_index[1]
    src_flat = jnp.concatenate([row, col + N])
    dst_flat = jnp.concatenate([col, row])

    hist = _deg_kernel(dst_flat).reshape(NC, NS, N_PAD)
    sc4 = _stats_kernel(hist)
    sc4 = sc4[:, :N].reshape(4, N, 1)

    h_inst = _mlp(x, enc_W1, enc_b1, enc_W2, enc_b2, blk=1000)
    h_net = _mlp(x_net, net_W1, net_b1, net_W2, net_b2, blk=1000)
    h = jnp.concatenate([h_inst, h_net], axis=0)

    h0 = h
    g2, selfsum = _prep(h0, conv_W[0], conv_b[0], conv_root[0],
                        reconv_W[0], reconv_b[0], reconv_root[0],
                        sc4, blk=1200)
    agg = _edge_kernel(g2.reshape(2 * N, D), src_flat, dst_flat)
    agg = agg.reshape(2, N_PAD, D)
    h1, g2, selfsum = _postprep(agg, sc4, selfsum, h0, ln_g[0], ln_b[0],
                                conv_W[1], conv_b[1], conv_root[1],
                                reconv_W[1], reconv_b[1], reconv_root[1],
                                leaky=True, blk=1200)
    agg = _edge_kernel(g2.reshape(2 * N, D), src_flat, dst_flat)
    agg = agg.reshape(2, N_PAD, D)
    h2 = _post(agg, sc4, selfsum, h1, ln_g[1], ln_b[1], leaky=False, blk=1200)
    return jnp.concatenate([h0, h1, h2], axis=1)


# speedup vs baseline: 1.0656x; 1.0068x over previous
import functools

import jax
import jax.numpy as jnp
from jax import lax
from jax.experimental import pallas as pl
from jax.experimental.pallas import tpu as pltpu
from jax.experimental.pallas import tpu_sc as plsc

N_INST = 10000
N_NET = 2000
N = N_INST + N_NET
E = 320000
D = 128
NC = 2
NS = 16
N_PAD = 12288
ROWS_PER_TILE = N_PAD // NS
CHUNK = 80
E_PER_W = E // NS
N_CHUNKS = E_PER_W // CHUNK
E_DIR = E
ZROWS = 16


def _leaky(v):
    return jnp.where(v >= 0, v, 0.1 * v)


def _deg_body(dst_hbm, out_hbm, hist_v, idx_v):
    c = lax.axis_index("c")
    s = lax.axis_index("s")
    zeros16 = jnp.zeros((16,), jnp.float32)
    ones16 = jnp.ones((16,), jnp.float32)

    def zero_step(i, _):
        hist_v[pl.ds(i * 16, 16)] = zeros16
        return 0

    lax.fori_loop(0, N_PAD // 16, zero_step, 0)

    pltpu.sync_copy(dst_hbm.at[pl.ds(c * E_DIR + s * E_PER_W, E_PER_W)], idx_v)

    def inner(j, _):
        idx = idx_v[pl.ds(j * 16, 16)]
        plsc.addupdate_scatter(hist_v, [idx], ones16)
        return 0

    lax.fori_loop(0, E_PER_W // 16, inner, 0)
    pltpu.sync_copy(hist_v, out_hbm.at[pl.ds((c * NS + s) * N_PAD, N_PAD)])


@jax.jit
def _deg_kernel(dst_flat):
    mesh = plsc.VectorSubcoreMesh(core_axis_name="c", subcore_axis_name="s")
    return pl.kernel(
        _deg_body,
        out_type=jax.ShapeDtypeStruct((NC * NS * N_PAD,), jnp.float32),
        mesh=mesh,
        scratch_types=[
            pltpu.VMEM((N_PAD,), jnp.float32),
            pltpu.VMEM((E_PER_W,), jnp.int32),
        ],
        compiler_params=pltpu.CompilerParams(needs_layout_passes=False),
    )(dst_flat)


def _edge_body(g2_hbm, src_hbm, dst_hbm, out_hbm, srcA_v, srcB_v, dstA_v,
               dstB_v, rows0_v, rows1_v, zero_v, acc_sh, gsem0, gsem1,
               isem0, isem1, zsem):
    c = lax.axis_index("c")
    s = lax.axis_index("s")
    zeros16 = jnp.zeros((16,), jnp.float32)
    lane = lax.iota(jnp.int32, 16)

    def zero_step(i, _):
        r = jnp.full((16,), i // (D // 16), jnp.int32)
        col = (i % (D // 16)) * 16 + lane
        plsc.store_scatter(zero_v, [r, col], zeros16)
        return 0

    lax.fori_loop(0, ZROWS * (D // 16), zero_step, 0)

    def zfill(i, _):
        pltpu.async_copy(
            zero_v, acc_sh.at[pl.ds(s * ROWS_PER_TILE + i * ZROWS, ZROWS)], zsem)
        return 0

    lax.fori_loop(0, ROWS_PER_TILE // ZROWS, zfill, 0)

    def zdrain(i, _):
        pltpu.make_async_copy(
            zero_v, acc_sh.at[pl.ds(s * ROWS_PER_TILE + i * ZROWS, ZROWS)], zsem
        ).wait()
        return 0

    lax.fori_loop(0, ROWS_PER_TILE // ZROWS, zdrain, 0)
    plsc.subcore_barrier()

    base0 = c * E_DIR + s * E_PER_W
    rows = (rows0_v, rows1_v)
    srcs = (srcA_v, srcB_v)
    dsts = (dstA_v, dstB_v)
    gsems = (gsem0, gsem1)
    isems = (isem0, isem1)

    def stage_idx(chunk, b):
        pltpu.async_copy(src_hbm.at[pl.ds(base0 + chunk * CHUNK, CHUNK)],
                         srcs[b], isems[b])
        pltpu.async_copy(dst_hbm.at[pl.ds(base0 + chunk * CHUNK, CHUNK)],
                         dsts[b], isems[b])

    def wait_idx(b):
        pltpu.make_async_copy(src_hbm.at[pl.ds(0, CHUNK)], srcs[b], isems[b]).wait()
        pltpu.make_async_copy(dst_hbm.at[pl.ds(0, CHUNK)], dsts[b], isems[b]).wait()

    pltpu.sync_copy(src_hbm.at[pl.ds(base0, CHUNK)], srcA_v)
    pltpu.sync_copy(dst_hbm.at[pl.ds(base0, CHUNK)], dstA_v)
    pltpu.async_copy(g2_hbm.at[srcA_v], rows0_v, gsem0)
    stage_idx(1, 1)

    def outer(p, _):
        for b in range(2):
            i = p * 2 + b
            nb = 1 - b
            wait_idx(nb)
            pltpu.async_copy(g2_hbm.at[srcs[nb]], rows[nb], gsems[nb])
            pltpu.make_async_copy(g2_hbm.at[srcs[b]], rows[b], gsems[b]).wait()
            pltpu.sync_copy(rows[b], acc_sh.at[dsts[b]], add=True)
            nxt = jnp.minimum(i + 2, N_CHUNKS - 1)
            stage_idx(nxt, b)
        return 0

    lax.fori_loop(0, N_CHUNKS // 2, outer, 0)
    pltpu.make_async_copy(g2_hbm.at[srcA_v], rows0_v, gsem0).wait()
    wait_idx(1)
    plsc.subcore_barrier()

    pltpu.sync_copy(
        acc_sh.at[pl.ds(s * ROWS_PER_TILE, ROWS_PER_TILE)],
        out_hbm.at[pl.ds(c * N_PAD + s * ROWS_PER_TILE, ROWS_PER_TILE)],
    )


@jax.jit
def _edge_kernel(g2, src_flat, dst_flat):
    mesh = plsc.VectorSubcoreMesh(core_axis_name="c", subcore_axis_name="s")
    return pl.kernel(
        _edge_body,
        out_type=jax.ShapeDtypeStruct((NC * N_PAD, D), jnp.float32),
        mesh=mesh,
        scratch_types=[
            pltpu.VMEM((CHUNK,), jnp.int32),
            pltpu.VMEM((CHUNK,), jnp.int32),
            pltpu.VMEM((CHUNK,), jnp.int32),
            pltpu.VMEM((CHUNK,), jnp.int32),
            pltpu.VMEM((CHUNK, D), jnp.float32),
            pltpu.VMEM((CHUNK, D), jnp.float32),
            pltpu.VMEM((ZROWS, D), jnp.float32),
            pltpu.VMEM_SHARED((N_PAD, D), jnp.float32),
            pltpu.SemaphoreType.DMA,
            pltpu.SemaphoreType.DMA,
            pltpu.SemaphoreType.DMA,
            pltpu.SemaphoreType.DMA,
            pltpu.SemaphoreType.DMA,
        ],
        compiler_params=pltpu.CompilerParams(needs_layout_passes=False),
    )(g2, src_flat, dst_flat)


def _mlp_body(x_ref, w1_ref, b1_ref, w2_ref, b2_ref, o_ref):
    h1 = _leaky(
        jnp.dot(x_ref[...], w1_ref[...], preferred_element_type=jnp.float32)
        + b1_ref[...]
    )
    o_ref[...] = _leaky(
        jnp.dot(h1, w2_ref[...], preferred_element_type=jnp.float32) + b2_ref[...]
    )


def _mlp(x, w1, b1, w2, b2, blk):
    m, _ = x.shape
    d_out = w2.shape[1]
    grid = m // blk
    return pl.pallas_call(
        _mlp_body,
        grid=(grid,),
        in_specs=[
            pl.BlockSpec((blk, x.shape[1]), lambda i: (i, 0)),
            pl.BlockSpec(w1.shape, lambda i: (0, 0)),
            pl.BlockSpec((1, b1.shape[0]), lambda i: (0, 0)),
            pl.BlockSpec(w2.shape, lambda i: (0, 0)),
            pl.BlockSpec((1, b2.shape[0]), lambda i: (0, 0)),
        ],
        out_specs=pl.BlockSpec((blk, d_out), lambda i: (i, 0)),
        out_shape=jax.ShapeDtypeStruct((m, d_out), jnp.float32),
    )(x, w1, b1[None, :], w2, b2[None, :])


def _stats_body(hist_ref, o_ref):
    deg_col = jnp.sum(hist_ref[0], axis=0, keepdims=True) + 1.0
    deg_row = jnp.sum(hist_ref[1], axis=0, keepdims=True) + 1.0
    o_ref[0:1] = lax.rsqrt(deg_row)
    o_ref[1:2] = 1.0 / deg_row
    o_ref[2:3] = lax.rsqrt(deg_col)
    o_ref[3:4] = 1.0 / deg_col


@jax.jit
def _stats_kernel(hist):
    return pl.pallas_call(
        _stats_body,
        out_shape=jax.ShapeDtypeStruct((4, N_PAD), jnp.float32),
    )(hist)


def _prep_body(h_ref, wf_ref, bf_ref, rootf_ref, wr_ref, br_ref, rootr_ref,
               sc_ref, g2_ref, self_ref):
    h = h_ref[...]
    dis_row = sc_ref[0, :, :]
    inv_row = sc_ref[1, :, :]
    dis_col = sc_ref[2, :, :]
    inv_col = sc_ref[3, :, :]
    xf = jnp.dot(h, wf_ref[...], preferred_element_type=jnp.float32) + bf_ref[...]
    xr = jnp.dot(h, wr_ref[...], preferred_element_type=jnp.float32) + br_ref[...]
    g2_ref[0] = dis_row * jax.nn.relu(xf)
    g2_ref[1] = dis_col * jax.nn.relu(xr)
    self_ref[...] = (
        inv_row * jax.nn.relu(xf + rootf_ref[...])
        + inv_col * jax.nn.relu(xr + rootr_ref[...])
    )


def _prep(h, wf, bf, rootf, wr, br, rootr, sc4, blk):
    grid = N // blk
    g2, selfsum = pl.pallas_call(
        _prep_body,
        grid=(grid,),
        in_specs=[
            pl.BlockSpec((blk, D), lambda i: (i, 0)),
            pl.BlockSpec((D, D), lambda i: (0, 0)),
            pl.BlockSpec((1, D), lambda i: (0, 0)),
            pl.BlockSpec((1, D), lambda i: (0, 0)),
            pl.BlockSpec((D, D), lambda i: (0, 0)),
            pl.BlockSpec((1, D), lambda i: (0, 0)),
            pl.BlockSpec((1, D), lambda i: (0, 0)),
            pl.BlockSpec((4, blk, 1), lambda i: (0, i, 0)),
        ],
        out_specs=[
            pl.BlockSpec((2, blk, D), lambda i: (0, i, 0)),
            pl.BlockSpec((blk, D), lambda i: (i, 0)),
        ],
        out_shape=[
            jax.ShapeDtypeStruct((2, N, D), jnp.float32),
            jax.ShapeDtypeStruct((N, D), jnp.float32),
        ],
    )(h, wf, bf[None, :], rootf[None, :], wr, br[None, :], rootr[None, :], sc4)
    return g2, selfsum


def _post_body(aggf_ref, aggr_ref, sc_ref, self_ref, h_ref, g_ref, b_ref,
               o_ref, *, leaky):
    dis_row = sc_ref[0, :, :]
    dis_col = sc_ref[2, :, :]
    hh = dis_row * aggf_ref[0] + dis_col * aggr_ref[0] + self_ref[...]
    mu = jnp.mean(hh, axis=-1, keepdims=True)
    var = jnp.mean(hh * hh, axis=-1, keepdims=True) - mu * mu
    hh = (hh - mu) * lax.rsqrt(var + 1e-5) * g_ref[...] + b_ref[...]
    if leaky:
        hh = _leaky(hh)
    o_ref[...] = hh + h_ref[...]


def _post(agg, sc4, selfsum, h, ln_g, ln_b, leaky, blk):
    grid = N // blk
    return pl.pallas_call(
        functools.partial(_post_body, leaky=leaky),
        grid=(grid,),
        in_specs=[
            pl.BlockSpec((1, blk, D), lambda i: (0, i, 0)),
            pl.BlockSpec((1, blk, D), lambda i: (1, i, 0)),
            pl.BlockSpec((4, blk, 1), lambda i: (0, i, 0)),
            pl.BlockSpec((blk, D), lambda i: (i, 0)),
            pl.BlockSpec((blk, D), lambda i: (i, 0)),
            pl.BlockSpec((1, D), lambda i: (0, 0)),
            pl.BlockSpec((1, D), lambda i: (0, 0)),
        ],
        out_specs=pl.BlockSpec((blk, D), lambda i: (i, 0)),
        out_shape=jax.ShapeDtypeStruct((N, D), jnp.float32),
    )(agg, agg, sc4, selfsum, h, ln_g[None, :], ln_b[None, :])


def _postprep_body(aggf_ref, aggr_ref, sc_ref, self_ref, h_ref, g_ref, b_ref,
                   wf_ref, bf_ref, rootf_ref, wr_ref, br_ref, rootr_ref,
                   o_ref, g2_ref, self2_ref, *, leaky):
    dis_row = sc_ref[0, :, :]
    inv_row = sc_ref[1, :, :]
    dis_col = sc_ref[2, :, :]
    inv_col = sc_ref[3, :, :]
    hh = dis_row * aggf_ref[0] + dis_col * aggr_ref[0] + self_ref[...]
    mu = jnp.mean(hh, axis=-1, keepdims=True)
    var = jnp.mean(hh * hh, axis=-1, keepdims=True) - mu * mu
    hh = (hh - mu) * lax.rsqrt(var + 1e-5) * g_ref[...] + b_ref[...]
    if leaky:
        hh = _leaky(hh)
    h = hh + h_ref[...]
    o_ref[...] = h
    xf = jnp.dot(h, wf_ref[...], preferred_element_type=jnp.float32) + bf_ref[...]
    xr = jnp.dot(h, wr_ref[...], preferred_element_type=jnp.float32) + br_ref[...]
    g2_ref[0] = dis_row * jax.nn.relu(xf)
    g2_ref[1] = dis_col * jax.nn.relu(xr)
    self2_ref[...] = (
        inv_row * jax.nn.relu(xf + rootf_ref[...])
        + inv_col * jax.nn.relu(xr + rootr_ref[...])
    )


def _postprep(agg, sc4, selfsum, h, ln_g, ln_b, wf, bf, rootf, wr, br, rootr,
              leaky, blk):
    grid = N // blk
    full = lambda i: (0, 0)
    return pl.pallas_call(
        functools.partial(_postprep_body, leaky=leaky),
        grid=(grid,),
        in_specs=[
            pl.BlockSpec((1, blk, D), lambda i: (0, i, 0)),
            pl.BlockSpec((1, blk, D), lambda i: (1, i, 0)),
            pl.BlockSpec((4, blk, 1), lambda i: (0, i, 0)),
            pl.BlockSpec((blk, D), lambda i: (i, 0)),
            pl.BlockSpec((blk, D), lambda i: (i, 0)),
            pl.BlockSpec((1, D), full),
            pl.BlockSpec((1, D), full),
            pl.BlockSpec((D, D), full),
            pl.BlockSpec((1, D), full),
            pl.BlockSpec((1, D), full),
            pl.BlockSpec((D, D), full),
            pl.BlockSpec((1, D), full),
            pl.BlockSpec((1, D), full),
        ],
        out_specs=[
            pl.BlockSpec((blk, D), lambda i: (i, 0)),
            pl.BlockSpec((2, blk, D), lambda i: (0, i, 0)),
            pl.BlockSpec((blk, D), lambda i: (i, 0)),
        ],
        out_shape=[
            jax.ShapeDtypeStruct((N, D), jnp.float32),
            jax.ShapeDtypeStruct((2, N, D), jnp.float32),
            jax.ShapeDtypeStruct((N, D), jnp.float32),
        ],
    )(agg, agg, sc4, selfsum, h, ln_g[None, :], ln_b[None, :], wf, bf[None, :],
      rootf[None, :], wr, br[None, :], rootr[None, :])


@jax.jit
def kernel(x, x_net, edge_index, enc_W1, enc_b1, enc_W2, enc_b2, net_W1,
           net_b1, net_W2, net_b2, conv_W, conv_b, conv_root, reconv_W,
           reconv_b, reconv_root, ln_g, ln_b):
    row = edge_index[0]
    col = edge_index[1]
    src_flat = jnp.concatenate([row, col + N])
    dst_flat = jnp.concatenate([col, row])

    hist = _deg_kernel(dst_flat).reshape(NC, NS, N_PAD)
    sc4 = _stats_kernel(hist)
    sc4 = sc4[:, :N].reshape(4, N, 1)

    h_inst = _mlp(x, enc_W1, enc_b1, enc_W2, enc_b2, blk=1000)
    h_net = _mlp(x_net, net_W1, net_b1, net_W2, net_b2, blk=1000)
    h = jnp.concatenate([h_inst, h_net], axis=0)

    h0 = h
    g2, selfsum = _prep(h0, conv_W[0], conv_b[0], conv_root[0],
                        reconv_W[0], reconv_b[0], reconv_root[0],
                        sc4, blk=2000)
    agg = _edge_kernel(g2.reshape(2 * N, D), src_flat, dst_flat)
    agg = agg.reshape(2, N_PAD, D)
    h1, g2, selfsum = _postprep(agg, sc4, selfsum, h0, ln_g[0], ln_b[0],
                                conv_W[1], conv_b[1], conv_root[1],
                                reconv_W[1], reconv_b[1], reconv_root[1],
                                leaky=True, blk=2000)
    agg = _edge_kernel(g2.reshape(2 * N, D), src_flat, dst_flat)
    agg = agg.reshape(2, N_PAD, D)
    h2 = _post(agg, sc4, selfsum, h1, ln_g[1], ln_b[1], leaky=False, blk=2000)
    return jnp.concatenate([h0, h1, h2], axis=1)
